# baseline (speedup 1.0000x reference)
import functools

import jax
import jax.numpy as jnp
from jax import lax
from jax.experimental import pallas as pl
from jax.experimental.pallas import tpu as pltpu
from jax.experimental.pallas import tpu_sc as plsc

N_ROWS_TBL = 1000000
D = 64
CHUNK = 128
NBUF = 5
N_WAVES = 2


def _transpose_in(wt):
    blk = 16384

    def body(x_ref, o_ref):
        o_ref[:, :D] = x_ref[...].T

    return pl.pallas_call(
        body,
        grid=(pl.cdiv(N_ROWS_TBL, blk),),
        in_specs=[pl.BlockSpec((D, blk), lambda g: (0, g))],
        out_specs=pl.BlockSpec((blk, 2 * D), lambda g: (g, 0)),
        out_shape=jax.ShapeDtypeStruct((N_ROWS_TBL, 2 * D), jnp.float32),
    )(wt)


def _split_out_wave(g2w, prev, wave, n_b, n_s):
    half = n_b // 2
    n_sw = n_s // N_WAVES
    s_off = wave * n_sw

    def body(x_ref, *refs):
        o_ref = refs[-1]
        x = x_ref[...]
        o_ref[0, :, pl.ds(0, half)] = x[:, :D].T
        o_ref[0, :, pl.ds(half, half)] = x[:, D:].T

    out_spec = pl.BlockSpec((1, D, n_b), lambda s: (s_off + s, 0, 0))
    in_specs = [pl.BlockSpec((half, 2 * D), lambda s: (s, 0))]
    operands = [g2w]
    kwargs = {}
    if prev is not None:
        in_specs.append(pl.BlockSpec(memory_space=pl.ANY))
        operands.append(prev)
        kwargs["input_output_aliases"] = {1: 0}

    return pl.pallas_call(
        body,
        grid=(n_sw,),
        in_specs=in_specs,
        out_specs=out_spec,
        out_shape=jax.ShapeDtypeStruct((n_s, D, n_b), jnp.float32),
        **kwargs,
    )(*operands)


@functools.lru_cache(maxsize=None)
def _make_gather(B, n_b, wave):
    info = plsc.get_sparse_core_info()
    NC, NS = info.num_cores, info.num_subcores
    NW = NC * NS
    Bw = B // N_WAVES
    assert Bw % (NW * CHUNK) == 0
    half = n_b // 2
    assert half % CHUNK == 0
    n_chunks = Bw // NW // CHUNK
    assert n_chunks % NBUF == 0 and n_chunks // NBUF >= 2
    n_outer = n_chunks // NBUF
    wave_chunk0 = wave * (Bw // CHUNK)

    mesh = plsc.VectorSubcoreMesh(core_axis_name="c", subcore_axis_name="s")

    scratch = [pltpu.VMEM((n_chunks, CHUNK), jnp.int32)]
    scratch += [pltpu.VMEM((CHUNK, 2 * D), jnp.float32) for _ in range(NBUF)]
    scratch += [pltpu.SemaphoreType.DMA for _ in range(2 * NBUF)]

    @functools.partial(
        pl.kernel,
        mesh=mesh,
        out_type=jax.ShapeDtypeStruct((Bw // 2, 2 * D), jnp.float32),
        scratch_types=scratch,
        compiler_params=pltpu.CompilerParams(use_tc_tiling_on_sc=False),
    )
    def gather_kernel(idx_hbm, table_hbm, out_hbm, idx_v, *rest):
        bufs = rest[:NBUF]
        gsems = rest[NBUF:2 * NBUF]
        wsems = rest[2 * NBUF:3 * NBUF]
        wid = lax.axis_index("s") * NC + lax.axis_index("c")
        chunk0 = wid * n_chunks

        pltpu.sync_copy(idx_hbm.at[wid], idx_v)

        def start_gather(j, b):
            pltpu.make_async_copy(
                table_hbm.at[idx_v.at[j]], bufs[b], gsems[b]).start()

        def wait_gather(j, b):
            pltpu.make_async_copy(
                table_hbm.at[idx_v.at[j]], bufs[b], gsems[b]).wait()

        def writeback(j, b):
            r0 = (wave_chunk0 + chunk0 + j) * CHUNK
            s = r0 // n_b
            b0 = r0 % n_b
            u0 = (s - wave * (Bw // n_b)) * half + (b0 % half)
            col0 = (b0 // half) * D
            return pltpu.make_async_copy(
                bufs[b].at[:, :D],
                out_hbm.at[pl.ds(u0, CHUNK), pl.ds(col0, D)],
                wsems[b])

        for b in range(NBUF):
            start_gather(b, b)

        def body(g, carry):
            for b in range(NBUF):
                j = g * NBUF + b
                wait_gather(j, b)
                w = writeback(j, b)
                w.start()
                w.wait()
                start_gather(j + NBUF, b)
            return carry

        lax.fori_loop(0, n_outer - 1, body, 0, unroll=False)

        for b in range(NBUF):
            j = (n_outer - 1) * NBUF + b
            wait_gather(j, b)
            writeback(j, b).start()
        for b in range(NBUF):
            j = (n_outer - 1) * NBUF + b
            writeback(j, b).wait()

    def run(idx_wave, table):
        idx3 = idx_wave.reshape(NW, n_chunks, CHUNK)
        return gather_kernel(idx3, table)

    return run


def kernel(Xb, weight):
    n_b, n_s = Xb.shape
    B = n_b * n_s
    Bw = B // N_WAVES
    table = _transpose_in(weight.T)
    idx_t = Xb.T.reshape(B).astype(jnp.int32)
    g2 = [
        _make_gather(B, n_b, w)(idx_t[w * Bw:(w + 1) * Bw], table)
        for w in range(N_WAVES)
    ]
    out = None
    for w in range(N_WAVES):
        out = _split_out_wave(g2[w], out, w, n_b, n_s)
    return jnp.transpose(out, (2, 0, 1))


# speedup vs baseline: 1.0072x; 1.0072x over previous
import functools

import jax
import jax.numpy as jnp
from jax import lax
from jax.experimental import pallas as pl
from jax.experimental.pallas import tpu as pltpu
from jax.experimental.pallas import tpu_sc as plsc

N_ROWS_TBL = 1000000
D = 64
CHUNK = 128
NBUF = 5
N_WAVES = 5


def _transpose_in(wt):
    blk = 16384

    def body(x_ref, o_ref):
        o_ref[:, :D] = x_ref[...].T

    return pl.pallas_call(
        body,
        grid=(pl.cdiv(N_ROWS_TBL, blk),),
        in_specs=[pl.BlockSpec((D, blk), lambda g: (0, g))],
        out_specs=pl.BlockSpec((blk, 2 * D), lambda g: (g, 0)),
        out_shape=jax.ShapeDtypeStruct((N_ROWS_TBL, 2 * D), jnp.float32),
    )(wt)


def _split_out_wave(g2w, prev, wave, n_b, n_s):
    half = n_b // 2
    n_sw = n_s // N_WAVES
    s_off = wave * n_sw

    def body(x_ref, *refs):
        o_ref = refs[-1]
        x = x_ref[...]
        o_ref[0, :, pl.ds(0, half)] = x[:, :D].T
        o_ref[0, :, pl.ds(half, half)] = x[:, D:].T

    out_spec = pl.BlockSpec((1, D, n_b), lambda s: (s_off + s, 0, 0))
    in_specs = [pl.BlockSpec((half, 2 * D), lambda s: (s, 0))]
    operands = [g2w]
    kwargs = {}
    if prev is not None:
        in_specs.append(pl.BlockSpec(memory_space=pl.ANY))
        operands.append(prev)
        kwargs["input_output_aliases"] = {1: 0}

    return pl.pallas_call(
        body,
        grid=(n_sw,),
        in_specs=in_specs,
        out_specs=out_spec,
        out_shape=jax.ShapeDtypeStruct((n_s, D, n_b), jnp.float32),
        **kwargs,
    )(*operands)


@functools.lru_cache(maxsize=None)
def _make_gather(B, n_b, wave):
    info = plsc.get_sparse_core_info()
    NC, NS = info.num_cores, info.num_subcores
    NW = NC * NS
    Bw = B // N_WAVES
    assert Bw % (NW * CHUNK) == 0
    half = n_b // 2
    assert half % CHUNK == 0
    n_chunks = Bw // NW // CHUNK
    assert n_chunks % NBUF == 0 and n_chunks // NBUF >= 2
    n_outer = n_chunks // NBUF
    wave_chunk0 = wave * (Bw // CHUNK)

    mesh = plsc.VectorSubcoreMesh(core_axis_name="c", subcore_axis_name="s")

    scratch = [pltpu.VMEM((n_chunks, CHUNK), jnp.int32)]
    scratch += [pltpu.VMEM((CHUNK, 2 * D), jnp.float32) for _ in range(NBUF)]
    scratch += [pltpu.SemaphoreType.DMA for _ in range(2 * NBUF)]

    @functools.partial(
        pl.kernel,
        mesh=mesh,
        out_type=jax.ShapeDtypeStruct((Bw // 2, 2 * D), jnp.float32),
        scratch_types=scratch,
        compiler_params=pltpu.CompilerParams(use_tc_tiling_on_sc=False),
    )
    def gather_kernel(idx_hbm, table_hbm, out_hbm, idx_v, *rest):
        bufs = rest[:NBUF]
        gsems = rest[NBUF:2 * NBUF]
        wsems = rest[2 * NBUF:3 * NBUF]
        wid = lax.axis_index("s") * NC + lax.axis_index("c")
        chunk0 = wid * n_chunks

        pltpu.sync_copy(idx_hbm.at[wid], idx_v)

        def start_gather(j, b):
            pltpu.make_async_copy(
                table_hbm.at[idx_v.at[j]], bufs[b], gsems[b]).start()

        def wait_gather(j, b):
            pltpu.make_async_copy(
                table_hbm.at[idx_v.at[j]], bufs[b], gsems[b]).wait()

        def writeback(j, b):
            r0 = (wave_chunk0 + chunk0 + j) * CHUNK
            s = r0 // n_b
            b0 = r0 % n_b
            u0 = (s - wave * (Bw // n_b)) * half + (b0 % half)
            col0 = (b0 // half) * D
            return pltpu.make_async_copy(
                bufs[b].at[:, :D],
                out_hbm.at[pl.ds(u0, CHUNK), pl.ds(col0, D)],
                wsems[b])

        for b in range(NBUF):
            start_gather(b, b)

        def body(g, carry):
            for b in range(NBUF):
                j = g * NBUF + b
                wait_gather(j, b)
                w = writeback(j, b)
                w.start()
                w.wait()
                start_gather(j + NBUF, b)
            return carry

        lax.fori_loop(0, n_outer - 1, body, 0, unroll=False)

        for b in range(NBUF):
            j = (n_outer - 1) * NBUF + b
            wait_gather(j, b)
            writeback(j, b).start()
        for b in range(NBUF):
            j = (n_outer - 1) * NBUF + b
            writeback(j, b).wait()

    def run(idx_wave, table):
        idx3 = idx_wave.reshape(NW, n_chunks, CHUNK)
        return gather_kernel(idx3, table)

    return run


def kernel(Xb, weight):
    n_b, n_s = Xb.shape
    B = n_b * n_s
    Bw = B // N_WAVES
    table = _transpose_in(weight.T)
    idx_t = Xb.T.reshape(B).astype(jnp.int32)
    g2 = [
        _make_gather(B, n_b, w)(idx_t[w * Bw:(w + 1) * Bw], table)
        for w in range(N_WAVES)
    ]
    out = None
    for w in range(N_WAVES):
        out = _split_out_wave(g2[w], out, w, n_b, n_s)
    return jnp.transpose(out, (2, 0, 1))


# speedup vs baseline: 1.0156x; 1.0084x over previous
import functools

import jax
import jax.numpy as jnp
from jax import lax
from jax.experimental import pallas as pl
from jax.experimental.pallas import tpu as pltpu
from jax.experimental.pallas import tpu_sc as plsc

N_ROWS_TBL = 1000000
D = 64
CHUNK = 128
NBUF = 5
N_WAVES = 5


def _transpose_in(wt):
    blk = 32768

    def body(x_ref, o_ref):
        o_ref[:, :D] = x_ref[...].T

    return pl.pallas_call(
        body,
        grid=(pl.cdiv(N_ROWS_TBL, blk),),
        in_specs=[pl.BlockSpec((D, blk), lambda g: (0, g))],
        out_specs=pl.BlockSpec((blk, 2 * D), lambda g: (g, 0)),
        out_shape=jax.ShapeDtypeStruct((N_ROWS_TBL, 2 * D), jnp.float32),
    )(wt)


def _split_out_wave(g2w, prev, wave, n_b, n_s):
    half = n_b // 2
    n_sw = n_s // N_WAVES
    s_off = wave * n_sw

    def body(x_ref, *refs):
        o_ref = refs[-1]
        x = x_ref[...]
        o_ref[0, :, pl.ds(0, half)] = x[:, :D].T
        o_ref[0, :, pl.ds(half, half)] = x[:, D:].T

    out_spec = pl.BlockSpec((1, D, n_b), lambda s: (s_off + s, 0, 0))
    in_specs = [pl.BlockSpec((half, 2 * D), lambda s: (s, 0))]
    operands = [g2w]
    kwargs = {}
    if prev is not None:
        in_specs.append(pl.BlockSpec(memory_space=pl.ANY))
        operands.append(prev)
        kwargs["input_output_aliases"] = {1: 0}

    return pl.pallas_call(
        body,
        grid=(n_sw,),
        in_specs=in_specs,
        out_specs=out_spec,
        out_shape=jax.ShapeDtypeStruct((n_s, D, n_b), jnp.float32),
        **kwargs,
    )(*operands)


@functools.lru_cache(maxsize=None)
def _make_gather(B, n_b, wave):
    info = plsc.get_sparse_core_info()
    NC, NS = info.num_cores, info.num_subcores
    NW = NC * NS
    Bw = B // N_WAVES
    assert Bw % (NW * CHUNK) == 0
    half = n_b // 2
    assert half % CHUNK == 0
    n_chunks = Bw // NW // CHUNK
    assert n_chunks % NBUF == 0 and n_chunks // NBUF >= 2
    n_outer = n_chunks // NBUF
    wave_chunk0 = wave * (Bw // CHUNK)

    mesh = plsc.VectorSubcoreMesh(core_axis_name="c", subcore_axis_name="s")

    scratch = [pltpu.VMEM((n_chunks, CHUNK), jnp.int32)]
    scratch += [pltpu.VMEM((CHUNK, 2 * D), jnp.float32) for _ in range(NBUF)]
    scratch += [pltpu.SemaphoreType.DMA for _ in range(2 * NBUF)]

    @functools.partial(
        pl.kernel,
        mesh=mesh,
        out_type=jax.ShapeDtypeStruct((Bw // 2, 2 * D), jnp.float32),
        scratch_types=scratch,
        compiler_params=pltpu.CompilerParams(use_tc_tiling_on_sc=False),
    )
    def gather_kernel(idx_hbm, table_hbm, out_hbm, idx_v, *rest):
        bufs = rest[:NBUF]
        gsems = rest[NBUF:2 * NBUF]
        wsems = rest[2 * NBUF:3 * NBUF]
        wid = lax.axis_index("s") * NC + lax.axis_index("c")
        chunk0 = wid * n_chunks

        pltpu.sync_copy(idx_hbm.at[wid], idx_v)

        def start_gather(j, b):
            pltpu.make_async_copy(
                table_hbm.at[idx_v.at[j]], bufs[b], gsems[b]).start()

        def wait_gather(j, b):
            pltpu.make_async_copy(
                table_hbm.at[idx_v.at[j]], bufs[b], gsems[b]).wait()

        def writeback(j, b):
            r0 = (wave_chunk0 + chunk0 + j) * CHUNK
            s = r0 // n_b
            b0 = r0 % n_b
            u0 = (s - wave * (Bw // n_b)) * half + (b0 % half)
            col0 = (b0 // half) * D
            return pltpu.make_async_copy(
                bufs[b].at[:, :D],
                out_hbm.at[pl.ds(u0, CHUNK), pl.ds(col0, D)],
                wsems[b])

        for b in range(NBUF):
            start_gather(b, b)

        def body(g, carry):
            for b in range(NBUF):
                j = g * NBUF + b
                wait_gather(j, b)
                w = writeback(j, b)
                w.start()
                w.wait()
                start_gather(j + NBUF, b)
            return carry

        lax.fori_loop(0, n_outer - 1, body, 0, unroll=False)

        for b in range(NBUF):
            j = (n_outer - 1) * NBUF + b
            wait_gather(j, b)
            writeback(j, b).start()
        for b in range(NBUF):
            j = (n_outer - 1) * NBUF + b
            writeback(j, b).wait()

    def run(idx_wave, table):
        idx3 = idx_wave.reshape(NW, n_chunks, CHUNK)
        return gather_kernel(idx3, table)

    return run


def kernel(Xb, weight):
    n_b, n_s = Xb.shape
    B = n_b * n_s
    Bw = B // N_WAVES
    table = _transpose_in(weight.T)
    idx_t = Xb.T.reshape(B).astype(jnp.int32)
    g2 = [
        _make_gather(B, n_b, w)(idx_t[w * Bw:(w + 1) * Bw], table)
        for w in range(N_WAVES)
    ]
    out = None
    for w in range(N_WAVES):
        out = _split_out_wave(g2[w], out, w, n_b, n_s)
    return jnp.transpose(out, (2, 0, 1))
